# baseline (speedup 1.0000x reference)
import functools

import jax
import jax.numpy as jnp
from jax.experimental import pallas as pl
from jax.experimental.pallas import tpu as pltpu
from jax.experimental.pallas import tpu_sc as plsc

B = 4
S = 2048
DIM = 1024
SD = 128
SH = 256
E = 8
BLK = 128
NBLK = 96
PADS = NBLK * BLK

_SC_NC = 2
_SC_NS = 16
_NW = _SC_NC * _SC_NS
_RPW = (B * S) // _NW
_CH = 32
_NCH = _RPW // _CH


def _plan_body(tok_ref, gpos_ref, code_ref, be_ref, used_ref):
    tok = tok_ref[...]
    xh = tok.astype(jnp.uint32)
    xh = xh ^ (xh >> 16)
    xh = xh * jnp.uint32(2246822507)
    xh = xh ^ (xh >> 13)
    xh = xh * jnp.uint32(3266489909)
    xh = xh ^ (xh >> 16)
    e = (xh & jnp.uint32(E - 1)).astype(jnp.int32)

    eids = jax.lax.broadcasted_iota(jnp.int32, (B, S, E), 2)
    oh = (e[:, :, None] == eids).astype(jnp.int32)

    c = oh
    k = 1
    while k < S:
        sh = jnp.concatenate(
            [jnp.zeros((B, k, E), jnp.int32), c[:, : S - k, :]], axis=1)
        c = c + sh
        k *= 2
    counts = c[:, S - 1, :]
    pc = ((counts + (BLK - 1)) // BLK) * BLK

    colsum = jnp.sum(pc, axis=0, keepdims=True, dtype=jnp.int32)
    ec = jnp.concatenate([jnp.zeros((1, 1), jnp.int32), colsum[:, : E - 1]],
                         axis=1)
    k = 1
    while k < E:
        ec = ec + jnp.concatenate(
            [jnp.zeros((1, k), jnp.int32), ec[:, : E - k]], axis=1)
        k *= 2
    rp = jnp.concatenate([jnp.zeros((1, E), jnp.int32), pc[: B - 1, :]],
                         axis=0)
    k = 1
    while k < B:
        rp = rp + jnp.concatenate(
            [jnp.zeros((k, E), jnp.int32), rp[: B - k, :]], axis=0)
        k *= 2
    off = ec + rp
    total = ec[:, E - 1 : E] + colsum[:, E - 1 : E]

    sel_off = jnp.sum(oh * off[:, None, :], axis=2, dtype=jnp.int32)
    sel_cnt = jnp.sum(oh * c, axis=2, dtype=jnp.int32)
    gpos_ref[...] = sel_off + sel_cnt - 1

    siota = (jax.lax.broadcasted_iota(jnp.int32, (NBLK, BLK), 0) * BLK
             + jax.lax.broadcasted_iota(jnp.int32, (NBLK, BLK), 1))
    valid = jnp.zeros((NBLK, BLK), jnp.bool_)
    start = jnp.zeros((NBLK, BLK), jnp.bool_)
    biota = jax.lax.broadcasted_iota(jnp.int32, (1, NBLK), 1) * BLK
    acc = jnp.zeros((1, NBLK), jnp.int32)
    for j in range(E):
        for b in range(B):
            offv = off[b : b + 1, j : j + 1]
            cv = counts[b : b + 1, j : j + 1]
            valid = valid | ((siota >= offv) & (siota < offv + cv))
            start = start | ((cv > 0) & (siota == offv))
            cond = (cv > 0) & (offv <= biota)
            acc = jnp.maximum(acc,
                              jnp.where(cond, jnp.int32(j), jnp.int32(0)))
    one = jnp.int32(1)
    two = jnp.int32(2)
    zero = jnp.int32(0)
    code_ref[...] = jnp.where(start, one, jnp.where(valid, two, zero))
    be_ref[...] = acc
    used_ref[...] = (biota < total).astype(jnp.int32)


def _plan(tok32):
    return pl.pallas_call(
        _plan_body,
        out_shape=(
            jax.ShapeDtypeStruct((B, S), jnp.int32),
            jax.ShapeDtypeStruct((NBLK, BLK), jnp.int32),
            jax.ShapeDtypeStruct((1, NBLK), jnp.int32),
            jax.ShapeDtypeStruct((1, NBLK), jnp.int32),
        ),
    )(tok32)


def _shift_down(m, k, fill):
    pad = jnp.full((k, m.shape[1]), fill, m.dtype)
    return jnp.concatenate([pad, m[: m.shape[0] - k, :]], axis=0)


def _moe_body(be_s, used_s, code_ref, x_ref, win_ref, wsi_ref, wso_ref,
              wout_ref, dp_ref, out_ref, carry_ref):
    i = pl.program_id(0)

    @pl.when(i == 0)
    def _init():
        carry_ref[...] = jnp.zeros_like(carry_ref)

    @pl.when(used_s[i] > 0)
    def _compute():
        X = x_ref[0]
        u = jnp.dot(X, win_ref[0], preferred_element_type=jnp.float32)
        sh = jnp.dot(X, wsi_ref[0], preferred_element_type=jnp.float32)
        sh = (sh * jax.nn.sigmoid(sh)).astype(jnp.bfloat16)
        sel = jnp.dot(sh, wso_ref[0], preferred_element_type=jnp.float32)
        a_raw = sel[:, 0 * SD : 1 * SD]
        b_raw = sel[:, 1 * SD : 2 * SD]
        c_raw = sel[:, 2 * SD : 3 * SD]
        d_raw = sel[:, 3 * SD : 4 * SD]

        code = code_ref[0]
        valid = code > 0
        interior = code > 1
        a_eff = jnp.where(interior, jax.nn.sigmoid(a_raw), 0.0)
        v_eff = jnp.where(valid, jnp.tanh(b_raw) * u, 0.0)

        A, V = a_eff, v_eff
        k = 1
        while k < BLK:
            V = V + A * _shift_down(V, k, 0.0)
            A = A * _shift_down(A, k, 1.0)
            k *= 2
        carry = carry_ref[0:1, :]
        h = V + A * carry
        carry_ref[0:1, :] = h[BLK - 1 : BLK, :]

        dp = dp_ref[0]
        y = jnp.tanh(c_raw) * h + dp * jax.nn.sigmoid(d_raw) * u
        out_ref[0] = jnp.dot(
            y.astype(jnp.bfloat16), wout_ref[0],
            preferred_element_type=jnp.float32).astype(jnp.bfloat16)


def _moe(be_flat, used_flat, code3, x_sorted, W_in, W_sel_in, W_sel_out,
         W_out, dp3):
    def imap_x(i, be, used):
        return (jnp.where(used[i] > 0, i, NBLK - 1), i * 0, i * 0)

    def imap_w(i, be, used):
        return (be[i], i * 0, i * 0)

    grid_spec = pltpu.PrefetchScalarGridSpec(
        num_scalar_prefetch=2,
        grid=(NBLK,),
        in_specs=[
            pl.BlockSpec((1, BLK, 1), imap_x),
            pl.BlockSpec((1, BLK, DIM), imap_x),
            pl.BlockSpec((1, DIM, SD), imap_w),
            pl.BlockSpec((1, DIM, SH), imap_w),
            pl.BlockSpec((1, SH, 4 * SD), imap_w),
            pl.BlockSpec((1, SD, DIM), imap_w),
            pl.BlockSpec((1, 1, SD), imap_w),
        ],
        out_specs=pl.BlockSpec((1, BLK, DIM), imap_x),
        scratch_shapes=[pltpu.VMEM((8, SD), jnp.float32)],
    )
    return pl.pallas_call(
        _moe_body,
        grid_spec=grid_spec,
        out_shape=jax.ShapeDtypeStruct((NBLK, BLK, DIM), jnp.bfloat16),
        compiler_params=pltpu.CompilerParams(
            dimension_semantics=("arbitrary",)),
    )(be_flat, used_flat, code3, x_sorted, W_in, W_sel_in, W_sel_out,
      W_out, dp3)


def _sc_mesh():
    return plsc.VectorSubcoreMesh(core_axis_name="c", subcore_axis_name="s")


def _sc_scatter(x_flat, idx3):
    @functools.partial(
        pl.kernel,
        mesh=_sc_mesh(),
        out_type=jax.ShapeDtypeStruct((PADS, DIM // 2), jnp.int32),
        scratch_types=[
            pltpu.VMEM((_NCH, _CH), jnp.int32),
            pltpu.VMEM((_CH, DIM // 2), jnp.int32),
            pltpu.VMEM((_CH, DIM // 2), jnp.int32),
            pltpu.SemaphoreType.DMA,
            pltpu.SemaphoreType.DMA,
            pltpu.SemaphoreType.DMA,
            pltpu.SemaphoreType.DMA,
        ],
    )
    def k(x_hbm, idx_hbm, out_hbm, idxv, rv0, rv1, sr0, sr1, sw0, sw1):
        wid = jax.lax.axis_index("s") * _SC_NC + jax.lax.axis_index("c")
        pltpu.sync_copy(idx_hbm.at[wid], idxv)
        bufs = (rv0, rv1)
        srs = (sr0, sr1)
        sws = (sw0, sw1)
        base = wid * _RPW
        rd0 = pltpu.async_copy(x_hbm.at[pl.ds(base, _CH)], rv0, sr0)
        reads = [rd0, None]
        writes = [None, None]
        for j in range(_NCH):
            p = j % 2
            q = (j + 1) % 2
            reads[p].wait()
            if j + 1 < _NCH:
                if writes[q] is not None:
                    writes[q].wait()
                reads[q] = pltpu.async_copy(
                    x_hbm.at[pl.ds(base + (j + 1) * _CH, _CH)], bufs[q],
                    srs[q])
            writes[p] = pltpu.async_copy(
                bufs[p], out_hbm.at[idxv.at[jnp.int32(j)]], sws[p])
        writes[0].wait()
        writes[1].wait()

    return k(x_flat, idx3)


def _sc_gather(src_flat, idx3):
    @functools.partial(
        pl.kernel,
        mesh=_sc_mesh(),
        out_type=jax.ShapeDtypeStruct((B * S, DIM // 2), jnp.int32),
        scratch_types=[
            pltpu.VMEM((_NCH, _CH), jnp.int32),
            pltpu.VMEM((_CH, DIM // 2), jnp.int32),
            pltpu.VMEM((_CH, DIM // 2), jnp.int32),
            pltpu.SemaphoreType.DMA,
            pltpu.SemaphoreType.DMA,
            pltpu.SemaphoreType.DMA,
            pltpu.SemaphoreType.DMA,
        ],
    )
    def k(src_hbm, idx_hbm, out_hbm, idxv, rv0, rv1, sr0, sr1, sw0, sw1):
        wid = jax.lax.axis_index("s") * _SC_NC + jax.lax.axis_index("c")
        pltpu.sync_copy(idx_hbm.at[wid], idxv)
        bufs = (rv0, rv1)
        srs = (sr0, sr1)
        sws = (sw0, sw1)
        base = wid * _RPW
        rd0 = pltpu.async_copy(src_hbm.at[idxv.at[jnp.int32(0)]], rv0, sr0)
        reads = [rd0, None]
        writes = [None, None]
        for j in range(_NCH):
            p = j % 2
            q = (j + 1) % 2
            reads[p].wait()
            if j + 1 < _NCH:
                if writes[q] is not None:
                    writes[q].wait()
                reads[q] = pltpu.async_copy(
                    src_hbm.at[idxv.at[jnp.int32(j + 1)]], bufs[q], srs[q])
            writes[p] = pltpu.async_copy(
                bufs[p], out_hbm.at[pl.ds(base + j * _CH, _CH)], sws[p])
        writes[0].wait()
        writes[1].wait()

    return k(src_flat, idx3)


def kernel(x, token_ids, W_in, W_sel_in, W_sel_out, W_out, d_param):
    tok32 = token_ids.astype(jnp.int32)
    gpos, code, be, used = _plan(tok32)

    idx3 = gpos.reshape(_NW, _NCH, _CH)
    xb = jax.lax.bitcast_convert_type(
        x.astype(jnp.bfloat16).reshape(B * S, DIM // 2, 2), jnp.int32)
    x_sorted = _sc_scatter(xb, idx3)
    x_sorted = jax.lax.bitcast_convert_type(
        x_sorted, jnp.bfloat16).reshape(NBLK, BLK, DIM)

    out_sorted = _moe(
        be.reshape(NBLK),
        used.reshape(NBLK),
        code.reshape(NBLK, BLK, 1),
        x_sorted.reshape(NBLK, BLK, DIM),
        W_in.astype(jnp.bfloat16),
        W_sel_in.astype(jnp.bfloat16),
        W_sel_out.astype(jnp.bfloat16),
        W_out.astype(jnp.bfloat16),
        d_param.reshape(E, 1, SD),
    )

    ob = jax.lax.bitcast_convert_type(
        out_sorted.reshape(PADS, DIM // 2, 2), jnp.int32)
    out = _sc_gather(ob, idx3)
    out = jax.lax.bitcast_convert_type(out, jnp.bfloat16)
    return out.reshape(B, S, DIM).astype(jnp.float32)


# speedup vs baseline: 4.8374x; 4.8374x over previous
import functools

import jax
import jax.numpy as jnp
from jax.experimental import pallas as pl
from jax.experimental.pallas import tpu as pltpu
from jax.experimental.pallas import tpu_sc as plsc

B = 4
S = 2048
DIM = 1024
SD = 128
SH = 256
E = 8
BLK = 128
NBLK = 96
PADS = NBLK * BLK

_SC_NC = 2
_SC_NS = 16
_NW = _SC_NC * _SC_NS
_RPW = (B * S) // _NW
_CH = 32
_NCH = _RPW // _CH


NG = E * B


def _plan_body(tok_ref, gpos_ref, code_ref, be_ref, used_ref):
    tok = tok_ref[...]
    xh = tok.astype(jnp.uint32)
    xh = xh ^ (xh >> 16)
    xh = xh * jnp.uint32(2246822507)
    xh = xh ^ (xh >> 13)
    xh = xh * jnp.uint32(3266489909)
    xh = xh ^ (xh >> 16)
    e = (xh & jnp.uint32(E - 1)).astype(jnp.int32)

    oh = jnp.concatenate(
        [(e == jnp.int32(ex)).astype(jnp.int32) for ex in range(E)],
        axis=1)

    c = oh
    k = 1
    while k < S:
        c = c + jnp.concatenate(
            [jnp.zeros((k, NG), jnp.int32), c[: S - k, :]], axis=0)
        k *= 2
    counts = c[S - 1 : S, :]
    pc = ((counts + (BLK - 1)) // BLK) * BLK

    po = jnp.concatenate([jnp.zeros((1, 1), jnp.int32), pc[:, : NG - 1]],
                         axis=1)
    k = 1
    while k < NG:
        po = po + jnp.concatenate(
            [jnp.zeros((1, k), jnp.int32), po[:, : NG - k]], axis=1)
        k *= 2
    total = po[:, NG - 1 : NG] + pc[:, NG - 1 : NG]

    m = (oh * (po + c - 1)).astype(jnp.float32)
    jj = jax.lax.broadcasted_iota(jnp.int32, (NG, B), 0)
    bb = jax.lax.broadcasted_iota(jnp.int32, (NG, B), 1)
    selm = ((jj & jnp.int32(B - 1)) == bb).astype(jnp.float32)
    pos_f = jnp.dot(m, selm, preferred_element_type=jnp.float32,
                    precision=jax.lax.Precision.HIGHEST)
    gpos_ref[...] = (pos_f + 0.5).astype(jnp.int32)

    siota = (jax.lax.broadcasted_iota(jnp.int32, (NBLK, BLK), 0) * BLK
             + jax.lax.broadcasted_iota(jnp.int32, (NBLK, BLK), 1))
    valid = jnp.zeros((NBLK, BLK), jnp.bool_)
    start = jnp.zeros((NBLK, BLK), jnp.bool_)
    biota = jax.lax.broadcasted_iota(jnp.int32, (1, NBLK), 1) * BLK
    acc = jnp.zeros((1, NBLK), jnp.int32)
    for ex in range(E):
        for b in range(B):
            g = ex * B + b
            offv = po[:, g : g + 1]
            cv = counts[:, g : g + 1]
            valid = valid | ((siota >= offv) & (siota < offv + cv))
            start = start | ((cv > 0) & (siota == offv))
            cond = (cv > 0) & (offv <= biota)
            acc = jnp.maximum(acc,
                              jnp.where(cond, jnp.int32(ex), jnp.int32(0)))
    one = jnp.int32(1)
    two = jnp.int32(2)
    zero = jnp.int32(0)
    code_ref[...] = jnp.where(start, one, jnp.where(valid, two, zero))
    be_ref[...] = acc
    used_ref[...] = (biota < total).astype(jnp.int32)


def _plan(tok_t):
    return pl.pallas_call(
        _plan_body,
        out_shape=(
            jax.ShapeDtypeStruct((S, B), jnp.int32),
            jax.ShapeDtypeStruct((NBLK, BLK), jnp.int32),
            jax.ShapeDtypeStruct((1, NBLK), jnp.int32),
            jax.ShapeDtypeStruct((1, NBLK), jnp.int32),
        ),
    )(tok_t)


def _shift_down(m, k, fill):
    pad = jnp.full((k, m.shape[1]), fill, m.dtype)
    return jnp.concatenate([pad, m[: m.shape[0] - k, :]], axis=0)


def _moe_body(be_s, used_s, code_ref, x_ref, win_ref, wso_ref,
              wout_ref, dp_ref, out_ref, carry_ref):
    i = pl.program_id(0)

    @pl.when(i == 0)
    def _init():
        carry_ref[...] = jnp.zeros_like(carry_ref)

    @pl.when(used_s[i] > 0)
    def _compute():
        X = x_ref[0].astype(jnp.bfloat16)
        t = jnp.dot(X, win_ref[0], preferred_element_type=jnp.float32)
        u = t[:, :SD]
        sh = t[:, SD:]
        sh = (sh * jax.nn.sigmoid(sh)).astype(jnp.bfloat16)
        sel = jnp.dot(sh, wso_ref[0], preferred_element_type=jnp.float32)
        a_raw = sel[:, 0 * SD : 1 * SD]
        b_raw = sel[:, 1 * SD : 2 * SD]
        c_raw = sel[:, 2 * SD : 3 * SD]
        d_raw = sel[:, 3 * SD : 4 * SD]

        code = code_ref[0]
        valid = code > 0
        interior = code > 1
        a_eff = jnp.where(interior, jax.nn.sigmoid(a_raw), 0.0)
        v_eff = jnp.where(valid, jnp.tanh(b_raw) * u, 0.0)

        A, V = a_eff, v_eff
        k = 1
        while k < BLK:
            V = V + A * _shift_down(V, k, 0.0)
            A = A * _shift_down(A, k, 1.0)
            k *= 2
        carry = carry_ref[0:1, :]
        h = V + A * carry
        carry_ref[0:1, :] = h[BLK - 1 : BLK, :]

        dp = dp_ref[0]
        y = jnp.tanh(c_raw) * h + dp * jax.nn.sigmoid(d_raw) * u
        out_ref[0] = jnp.dot(y.astype(jnp.bfloat16), wout_ref[0],
                             preferred_element_type=jnp.float32)


def _moe(be_flat, used_flat, code3, x_sorted, W_c, W_sel_out, W_out, dp3):
    def imap_x(i, be, used):
        return (jnp.where(used[i] > 0, i, NBLK - 1), i * 0, i * 0)

    def imap_w(i, be, used):
        return (be[i], i * 0, i * 0)

    grid_spec = pltpu.PrefetchScalarGridSpec(
        num_scalar_prefetch=2,
        grid=(NBLK,),
        in_specs=[
            pl.BlockSpec((1, BLK, 1), imap_x),
            pl.BlockSpec((1, BLK, DIM), imap_x),
            pl.BlockSpec((1, DIM, SD + SH), imap_w),
            pl.BlockSpec((1, SH, 4 * SD), imap_w),
            pl.BlockSpec((1, SD, DIM), imap_w),
            pl.BlockSpec((1, 1, SD), imap_w),
        ],
        out_specs=pl.BlockSpec((1, BLK, DIM), imap_x),
        scratch_shapes=[pltpu.VMEM((8, SD), jnp.float32)],
    )
    return pl.pallas_call(
        _moe_body,
        grid_spec=grid_spec,
        out_shape=jax.ShapeDtypeStruct((NBLK, BLK, DIM), jnp.float32),
        compiler_params=pltpu.CompilerParams(
            dimension_semantics=("arbitrary",)),
    )(be_flat, used_flat, code3, x_sorted, W_c, W_sel_out, W_out, dp3)


def _sc_mesh():
    return plsc.VectorSubcoreMesh(core_axis_name="c", subcore_axis_name="s")


def _sc_scatter(x_flat, idx3):
    @functools.partial(
        pl.kernel,
        mesh=_sc_mesh(),
        out_type=jax.ShapeDtypeStruct((PADS, DIM), jnp.float32),
        scratch_types=[
            pltpu.VMEM((_NCH, _CH), jnp.int32),
            pltpu.VMEM((_CH, DIM), jnp.float32),
            pltpu.VMEM((_CH, DIM), jnp.float32),
            pltpu.SemaphoreType.DMA,
            pltpu.SemaphoreType.DMA,
            pltpu.SemaphoreType.DMA,
            pltpu.SemaphoreType.DMA,
        ],
    )
    def k(x_hbm, idx_hbm, out_hbm, idxv, rv0, rv1, sr0, sr1, sw0, sw1):
        wid = jax.lax.axis_index("s") * _SC_NC + jax.lax.axis_index("c")
        pltpu.sync_copy(idx_hbm.at[wid], idxv)
        bufs = (rv0, rv1)
        srs = (sr0, sr1)
        sws = (sw0, sw1)
        base = wid * _RPW
        rd0 = pltpu.async_copy(x_hbm.at[pl.ds(base, _CH)], rv0, sr0)
        reads = [rd0, None]
        writes = [None, None]
        for j in range(_NCH):
            p = j % 2
            q = (j + 1) % 2
            reads[p].wait()
            if j + 1 < _NCH:
                if writes[q] is not None:
                    writes[q].wait()
                reads[q] = pltpu.async_copy(
                    x_hbm.at[pl.ds(base + (j + 1) * _CH, _CH)], bufs[q],
                    srs[q])
            writes[p] = pltpu.async_copy(
                bufs[p], out_hbm.at[idxv.at[jnp.int32(j)]], sws[p])
        writes[0].wait()
        writes[1].wait()

    return k(x_flat, idx3)


def _sc_gather(src_flat, idx3):
    @functools.partial(
        pl.kernel,
        mesh=_sc_mesh(),
        out_type=jax.ShapeDtypeStruct((B * S, DIM), jnp.float32),
        scratch_types=[
            pltpu.VMEM((_NCH, _CH), jnp.int32),
            pltpu.VMEM((_CH, DIM), jnp.float32),
            pltpu.VMEM((_CH, DIM), jnp.float32),
            pltpu.SemaphoreType.DMA,
            pltpu.SemaphoreType.DMA,
            pltpu.SemaphoreType.DMA,
            pltpu.SemaphoreType.DMA,
        ],
    )
    def k(src_hbm, idx_hbm, out_hbm, idxv, rv0, rv1, sr0, sr1, sw0, sw1):
        wid = jax.lax.axis_index("s") * _SC_NC + jax.lax.axis_index("c")
        pltpu.sync_copy(idx_hbm.at[wid], idxv)
        bufs = (rv0, rv1)
        srs = (sr0, sr1)
        sws = (sw0, sw1)
        base = wid * _RPW
        rd0 = pltpu.async_copy(src_hbm.at[idxv.at[jnp.int32(0)]], rv0, sr0)
        reads = [rd0, None]
        writes = [None, None]
        for j in range(_NCH):
            p = j % 2
            q = (j + 1) % 2
            reads[p].wait()
            if j + 1 < _NCH:
                if writes[q] is not None:
                    writes[q].wait()
                reads[q] = pltpu.async_copy(
                    src_hbm.at[idxv.at[jnp.int32(j + 1)]], bufs[q], srs[q])
            writes[p] = pltpu.async_copy(
                bufs[p], out_hbm.at[pl.ds(base + j * _CH, _CH)], sws[p])
        writes[0].wait()
        writes[1].wait()

    return k(src_flat, idx3)


def kernel(x, token_ids, W_in, W_sel_in, W_sel_out, W_out, d_param):
    tok_t = token_ids.astype(jnp.int32).T
    gpos_t, code, be, used = _plan(tok_t)

    idx3 = gpos_t.T.reshape(_NW, _NCH, _CH)
    x_sorted = _sc_scatter(x.reshape(B * S, DIM), idx3)

    out_sorted = _moe(
        be.reshape(NBLK),
        used.reshape(NBLK),
        code.reshape(NBLK, BLK, 1),
        x_sorted.reshape(NBLK, BLK, DIM),
        jnp.concatenate([W_in, W_sel_in], axis=2).astype(jnp.bfloat16),
        W_sel_out.astype(jnp.bfloat16),
        W_out.astype(jnp.bfloat16),
        d_param.reshape(E, 1, SD),
    )

    out = _sc_gather(out_sorted.reshape(PADS, DIM), idx3)
    return out.reshape(B, S, DIM)


# speedup vs baseline: 4.8766x; 1.0081x over previous
import functools

import jax
import jax.numpy as jnp
from jax.experimental import pallas as pl
from jax.experimental.pallas import tpu as pltpu
from jax.experimental.pallas import tpu_sc as plsc

B = 4
S = 2048
DIM = 1024
SD = 128
SH = 256
E = 8
BLK = 128
NBLK = 96
PADS = NBLK * BLK

_SC_NC = 2
_SC_NS = 16
_NW = _SC_NC * _SC_NS
_RPW = (B * S) // _NW
_CH = 32
_NCH = _RPW // _CH


NG = E * B


def _plan_body(tok_ref, gpos_ref, code_ref, be_ref, used_ref):
    tok = tok_ref[...]
    xh = tok.astype(jnp.uint32)
    xh = xh ^ (xh >> 16)
    xh = xh * jnp.uint32(2246822507)
    xh = xh ^ (xh >> 13)
    xh = xh * jnp.uint32(3266489909)
    xh = xh ^ (xh >> 16)
    e = (xh & jnp.uint32(E - 1)).astype(jnp.int32)

    oh = jnp.concatenate(
        [(e == jnp.int32(ex)).astype(jnp.int32) for ex in range(E)],
        axis=1)

    c = oh
    k = 1
    while k < S:
        c = c + jnp.concatenate(
            [jnp.zeros((k, NG), jnp.int32), c[: S - k, :]], axis=0)
        k *= 2
    counts = c[S - 1 : S, :]
    pc = ((counts + (BLK - 1)) // BLK) * BLK

    po = jnp.concatenate([jnp.zeros((1, 1), jnp.int32), pc[:, : NG - 1]],
                         axis=1)
    k = 1
    while k < NG:
        po = po + jnp.concatenate(
            [jnp.zeros((1, k), jnp.int32), po[:, : NG - k]], axis=1)
        k *= 2
    total = po[:, NG - 1 : NG] + pc[:, NG - 1 : NG]

    m = (oh * (po + c - 1)).astype(jnp.float32)
    jj = jax.lax.broadcasted_iota(jnp.int32, (NG, B), 0)
    bb = jax.lax.broadcasted_iota(jnp.int32, (NG, B), 1)
    selm = ((jj & jnp.int32(B - 1)) == bb).astype(jnp.float32)
    pos_f = jnp.dot(m, selm, preferred_element_type=jnp.float32,
                    precision=jax.lax.Precision.HIGHEST)
    gpos_ref[...] = (pos_f + 0.5).astype(jnp.int32)

    siota = (jax.lax.broadcasted_iota(jnp.int32, (NBLK, BLK), 0) * BLK
             + jax.lax.broadcasted_iota(jnp.int32, (NBLK, BLK), 1))
    valid = jnp.zeros((NBLK, BLK), jnp.bool_)
    start = jnp.zeros((NBLK, BLK), jnp.bool_)
    biota = jax.lax.broadcasted_iota(jnp.int32, (1, NBLK), 1) * BLK
    acc = jnp.zeros((1, NBLK), jnp.int32)
    for ex in range(E):
        for b in range(B):
            g = ex * B + b
            offv = po[:, g : g + 1]
            cv = counts[:, g : g + 1]
            valid = valid | ((siota >= offv) & (siota < offv + cv))
            start = start | ((cv > 0) & (siota == offv))
            cond = (cv > 0) & (offv <= biota)
            acc = jnp.maximum(acc,
                              jnp.where(cond, jnp.int32(ex), jnp.int32(0)))
    one = jnp.int32(1)
    two = jnp.int32(2)
    zero = jnp.int32(0)
    code_ref[...] = jnp.where(start, one, jnp.where(valid, two, zero))
    be_ref[...] = acc
    used_ref[...] = (biota < total).astype(jnp.int32)


def _plan(tok_t):
    return pl.pallas_call(
        _plan_body,
        out_shape=(
            jax.ShapeDtypeStruct((S, B), jnp.int32),
            jax.ShapeDtypeStruct((NBLK, BLK), jnp.int32),
            jax.ShapeDtypeStruct((1, NBLK), jnp.int32),
            jax.ShapeDtypeStruct((1, NBLK), jnp.int32),
        ),
    )(tok_t)


def _shift_down(m, k, fill):
    pad = jnp.full((k, m.shape[1]), fill, m.dtype)
    return jnp.concatenate([pad, m[: m.shape[0] - k, :]], axis=0)


def _moe_body(be_s, used_s, code_ref, x_ref, win_ref, wso_ref,
              wout_ref, dp_ref, out_ref, carry_ref):
    i = pl.program_id(0)

    @pl.when(i == 0)
    def _init():
        carry_ref[...] = jnp.zeros_like(carry_ref)

    @pl.when(used_s[i] > 0)
    def _compute():
        ex = be_s[i]
        X = x_ref[0].astype(jnp.bfloat16)
        t = jnp.dot(X, win_ref[ex], preferred_element_type=jnp.float32)
        u = t[:, :SD]
        sh = t[:, SD:]
        sh = (sh * jax.nn.sigmoid(sh)).astype(jnp.bfloat16)
        sel = jnp.dot(sh, wso_ref[ex], preferred_element_type=jnp.float32)
        a_raw = sel[:, 0 * SD : 1 * SD]
        b_raw = sel[:, 1 * SD : 2 * SD]
        c_raw = sel[:, 2 * SD : 3 * SD]
        d_raw = sel[:, 3 * SD : 4 * SD]

        code = code_ref[0]
        valid = code > 0
        interior = code > 1
        a_eff = jnp.where(interior, jax.nn.sigmoid(a_raw), 0.0)
        v_eff = jnp.where(valid, jnp.tanh(b_raw) * u, 0.0)

        A, V = a_eff, v_eff
        k = 1
        while k < BLK:
            V = V + A * _shift_down(V, k, 0.0)
            A = A * _shift_down(A, k, 1.0)
            k *= 2
        carry = carry_ref[0:1, :]
        h = V + A * carry
        carry_ref[0:1, :] = h[BLK - 1 : BLK, :]

        dp = dp_ref[ex]
        y = jnp.tanh(c_raw) * h + dp * jax.nn.sigmoid(d_raw) * u
        out_ref[0] = jnp.dot(y.astype(jnp.bfloat16), wout_ref[ex],
                             preferred_element_type=jnp.float32)


def _moe(be_flat, used_flat, code3, x_sorted, W_c, W_sel_out, W_out, dp3):
    def imap_x(i, be, used):
        return (jnp.where(used[i] > 0, i, NBLK - 1), i * 0, i * 0)

    def imap_w(i, be, used):
        return (i * 0, i * 0, i * 0)

    grid_spec = pltpu.PrefetchScalarGridSpec(
        num_scalar_prefetch=2,
        grid=(NBLK,),
        in_specs=[
            pl.BlockSpec((1, BLK, 1), imap_x),
            pl.BlockSpec((1, BLK, DIM), imap_x),
            pl.BlockSpec((E, DIM, SD + SH), imap_w),
            pl.BlockSpec((E, SH, 4 * SD), imap_w),
            pl.BlockSpec((E, SD, DIM), imap_w),
            pl.BlockSpec((E, 1, SD), imap_w),
        ],
        out_specs=pl.BlockSpec((1, BLK, DIM), imap_x),
        scratch_shapes=[pltpu.VMEM((8, SD), jnp.float32)],
    )
    return pl.pallas_call(
        _moe_body,
        grid_spec=grid_spec,
        out_shape=jax.ShapeDtypeStruct((NBLK, BLK, DIM), jnp.float32),
        compiler_params=pltpu.CompilerParams(
            dimension_semantics=("arbitrary",)),
    )(be_flat, used_flat, code3, x_sorted, W_c, W_sel_out, W_out, dp3)


def _sc_mesh():
    return plsc.VectorSubcoreMesh(core_axis_name="c", subcore_axis_name="s")


def _sc_scatter(x_flat, idx3):
    @functools.partial(
        pl.kernel,
        mesh=_sc_mesh(),
        out_type=jax.ShapeDtypeStruct((PADS, DIM), jnp.float32),
        scratch_types=[
            pltpu.VMEM((_NCH, _CH), jnp.int32),
            pltpu.VMEM((_CH, DIM), jnp.float32),
            pltpu.VMEM((_CH, DIM), jnp.float32),
            pltpu.SemaphoreType.DMA,
            pltpu.SemaphoreType.DMA,
            pltpu.SemaphoreType.DMA,
            pltpu.SemaphoreType.DMA,
        ],
    )
    def k(x_hbm, idx_hbm, out_hbm, idxv, rv0, rv1, sr0, sr1, sw0, sw1):
        wid = jax.lax.axis_index("s") * _SC_NC + jax.lax.axis_index("c")
        pltpu.sync_copy(idx_hbm.at[wid], idxv)
        bufs = (rv0, rv1)
        srs = (sr0, sr1)
        sws = (sw0, sw1)
        base = wid * _RPW
        rd0 = pltpu.async_copy(x_hbm.at[pl.ds(base, _CH)], rv0, sr0)
        reads = [rd0, None]
        writes = [None, None]
        for j in range(_NCH):
            p = j % 2
            q = (j + 1) % 2
            reads[p].wait()
            if j + 1 < _NCH:
                if writes[q] is not None:
                    writes[q].wait()
                reads[q] = pltpu.async_copy(
                    x_hbm.at[pl.ds(base + (j + 1) * _CH, _CH)], bufs[q],
                    srs[q])
            writes[p] = pltpu.async_copy(
                bufs[p], out_hbm.at[idxv.at[jnp.int32(j)]], sws[p])
        writes[0].wait()
        writes[1].wait()

    return k(x_flat, idx3)


def _sc_gather(src_flat, idx3):
    @functools.partial(
        pl.kernel,
        mesh=_sc_mesh(),
        out_type=jax.ShapeDtypeStruct((B * S, DIM), jnp.float32),
        scratch_types=[
            pltpu.VMEM((_NCH, _CH), jnp.int32),
            pltpu.VMEM((_CH, DIM), jnp.float32),
            pltpu.VMEM((_CH, DIM), jnp.float32),
            pltpu.SemaphoreType.DMA,
            pltpu.SemaphoreType.DMA,
            pltpu.SemaphoreType.DMA,
            pltpu.SemaphoreType.DMA,
        ],
    )
    def k(src_hbm, idx_hbm, out_hbm, idxv, rv0, rv1, sr0, sr1, sw0, sw1):
        wid = jax.lax.axis_index("s") * _SC_NC + jax.lax.axis_index("c")
        pltpu.sync_copy(idx_hbm.at[wid], idxv)
        bufs = (rv0, rv1)
        srs = (sr0, sr1)
        sws = (sw0, sw1)
        base = wid * _RPW
        rd0 = pltpu.async_copy(src_hbm.at[idxv.at[jnp.int32(0)]], rv0, sr0)
        reads = [rd0, None]
        writes = [None, None]
        for j in range(_NCH):
            p = j % 2
            q = (j + 1) % 2
            reads[p].wait()
            if j + 1 < _NCH:
                if writes[q] is not None:
                    writes[q].wait()
                reads[q] = pltpu.async_copy(
                    src_hbm.at[idxv.at[jnp.int32(j + 1)]], bufs[q], srs[q])
            writes[p] = pltpu.async_copy(
                bufs[p], out_hbm.at[pl.ds(base + j * _CH, _CH)], sws[p])
        writes[0].wait()
        writes[1].wait()

    return k(src_flat, idx3)


def kernel(x, token_ids, W_in, W_sel_in, W_sel_out, W_out, d_param):
    tok_t = token_ids.astype(jnp.int32).T
    gpos_t, code, be, used = _plan(tok_t)

    idx3 = gpos_t.T.reshape(_NW, _NCH, _CH)
    x_sorted = _sc_scatter(x.reshape(B * S, DIM), idx3)

    out_sorted = _moe(
        be.reshape(NBLK),
        used.reshape(NBLK),
        code.reshape(NBLK, BLK, 1),
        x_sorted.reshape(NBLK, BLK, DIM),
        jnp.concatenate([W_in, W_sel_in], axis=2).astype(jnp.bfloat16),
        W_sel_out.astype(jnp.bfloat16),
        W_out.astype(jnp.bfloat16),
        d_param.reshape(E, 1, SD),
    )

    out = _sc_gather(out_sorted.reshape(PADS, DIM), idx3)
    return out.reshape(B, S, DIM)


# speedup vs baseline: 4.9771x; 1.0206x over previous
import functools

import jax
import jax.numpy as jnp
from jax.experimental import pallas as pl
from jax.experimental.pallas import tpu as pltpu
from jax.experimental.pallas import tpu_sc as plsc

B = 4
S = 2048
DIM = 1024
SD = 128
SH = 256
E = 8
BLK = 128
NBLK = 96
PADS = NBLK * BLK

_SC_NC = 2
_SC_NS = 16
_NW = _SC_NC * _SC_NS
_RPW = (B * S) // _NW
_CH = 32
_NCH = _RPW // _CH


NG = E * B


def _plan_body(tok_ref, gpos_ref, be_ref, used_ref, vcnt_ref, fstart_ref):
    tok = tok_ref[...]
    xh = tok.astype(jnp.uint32)
    xh = xh ^ (xh >> 16)
    xh = xh * jnp.uint32(2246822507)
    xh = xh ^ (xh >> 13)
    xh = xh * jnp.uint32(3266489909)
    xh = xh ^ (xh >> 16)
    e = (xh & jnp.uint32(E - 1)).astype(jnp.int32)

    oh = jnp.concatenate(
        [(e == jnp.int32(ex)).astype(jnp.int32) for ex in range(E)],
        axis=1)

    c = oh
    k = 1
    while k < S:
        c = c + jnp.concatenate(
            [jnp.zeros((k, NG), jnp.int32), c[: S - k, :]], axis=0)
        k *= 2
    counts = c[S - 1 : S, :]
    pc = ((counts + (BLK - 1)) // BLK) * BLK

    po = jnp.concatenate([jnp.zeros((1, 1), jnp.int32), pc[:, : NG - 1]],
                         axis=1)
    k = 1
    while k < NG:
        po = po + jnp.concatenate(
            [jnp.zeros((1, k), jnp.int32), po[:, : NG - k]], axis=1)
        k *= 2
    total = po[:, NG - 1 : NG] + pc[:, NG - 1 : NG]

    m = (oh * (po + c - 1)).astype(jnp.float32)
    jj = jax.lax.broadcasted_iota(jnp.int32, (NG, B), 0)
    bb = jax.lax.broadcasted_iota(jnp.int32, (NG, B), 1)
    selm = ((jj & jnp.int32(B - 1)) == bb).astype(jnp.float32)
    pos_f = jnp.dot(m, selm, preferred_element_type=jnp.float32,
                    precision=jax.lax.Precision.HIGHEST)
    gpos_ref[...] = (pos_f + 0.5).astype(jnp.int32)

    biota = jax.lax.broadcasted_iota(jnp.int32, (1, NBLK), 1) * BLK
    acc = jnp.zeros((1, NBLK), jnp.int32)
    vcnt = jnp.zeros((1, NBLK), jnp.int32)
    fstart = jnp.zeros((1, NBLK), jnp.int32)
    for ex in range(E):
        for b in range(B):
            g = ex * B + b
            offv = po[:, g : g + 1]
            cv = counts[:, g : g + 1]
            pcv = pc[:, g : g + 1]
            cond = (cv > 0) & (offv <= biota)
            acc = jnp.maximum(acc,
                              jnp.where(cond, jnp.int32(ex), jnp.int32(0)))
            in_group = (offv <= biota) & (biota < offv + pcv)
            vcnt = vcnt + jnp.where(
                in_group, jnp.clip(offv + cv - biota, 0, BLK), jnp.int32(0))
            fstart = fstart + jnp.where(
                in_group & (offv == biota), jnp.int32(1), jnp.int32(0))
    be_ref[...] = acc
    used_ref[...] = (biota < total).astype(jnp.int32)
    vcnt_ref[...] = vcnt
    fstart_ref[...] = fstart


def _plan(tok_t):
    return pl.pallas_call(
        _plan_body,
        out_shape=(
            jax.ShapeDtypeStruct((S, B), jnp.int32),
            jax.ShapeDtypeStruct((1, NBLK), jnp.int32),
            jax.ShapeDtypeStruct((1, NBLK), jnp.int32),
            jax.ShapeDtypeStruct((1, NBLK), jnp.int32),
            jax.ShapeDtypeStruct((1, NBLK), jnp.int32),
        ),
    )(tok_t)


def _shift_down(m, k, fill):
    pad = jnp.full((k, m.shape[1]), fill, m.dtype)
    return jnp.concatenate([pad, m[: m.shape[0] - k, :]], axis=0)


def _moe_body(be_s, used_s, vcnt_s, fstart_s, x_ref, win_ref, wso_ref,
              wout_ref, dp_ref, out_ref, carry_ref):
    i = pl.program_id(0)

    @pl.when(i == 0)
    def _init():
        carry_ref[...] = jnp.zeros_like(carry_ref)

    @pl.when(used_s[i] > 0)
    def _compute():
        ex = be_s[i]
        X = x_ref[0].astype(jnp.bfloat16)
        t = jnp.dot(X, win_ref[ex], preferred_element_type=jnp.float32)
        u = t[:, :SD]
        sh = t[:, SD:]
        sh = (sh * jax.nn.sigmoid(sh)).astype(jnp.bfloat16)
        sel = jnp.dot(sh, wso_ref[ex], preferred_element_type=jnp.float32)
        a_raw = sel[:, 0 * SD : 1 * SD]
        b_raw = sel[:, 1 * SD : 2 * SD]
        c_raw = sel[:, 2 * SD : 3 * SD]
        d_raw = sel[:, 3 * SD : 4 * SD]

        riota = jax.lax.broadcasted_iota(jnp.int32, (BLK, 1), 0)
        valid = riota < vcnt_s[i]
        interior = valid & ((riota > 0) | (fstart_s[i] == 0))
        a_eff = jnp.where(interior, jax.nn.sigmoid(a_raw), 0.0)
        v_eff = jnp.where(valid, jnp.tanh(b_raw) * u, 0.0)

        A, V = a_eff, v_eff
        k = 1
        while k < BLK:
            V = V + A * _shift_down(V, k, 0.0)
            A = A * _shift_down(A, k, 1.0)
            k *= 2
        carry = carry_ref[0:1, :]
        h = V + A * carry
        carry_ref[0:1, :] = h[BLK - 1 : BLK, :]

        dp = dp_ref[ex]
        y = jnp.tanh(c_raw) * h + dp * jax.nn.sigmoid(d_raw) * u
        out_ref[0] = jnp.dot(y.astype(jnp.bfloat16), wout_ref[ex],
                             preferred_element_type=jnp.float32)


def _moe(be_flat, used_flat, vcnt_flat, fstart_flat, x_sorted, W_c,
         W_sel_out, W_out, dp3):
    def imap_x(i, be, used, vcnt, fs):
        return (jnp.where(used[i] > 0, i, NBLK - 1), i * 0, i * 0)

    def imap_w(i, be, used, vcnt, fs):
        return (i * 0, i * 0, i * 0)

    grid_spec = pltpu.PrefetchScalarGridSpec(
        num_scalar_prefetch=4,
        grid=(NBLK,),
        in_specs=[
            pl.BlockSpec((1, BLK, DIM), imap_x),
            pl.BlockSpec((E, DIM, SD + SH), imap_w),
            pl.BlockSpec((E, SH, 4 * SD), imap_w),
            pl.BlockSpec((E, SD, DIM), imap_w),
            pl.BlockSpec((E, 1, SD), imap_w),
        ],
        out_specs=pl.BlockSpec((1, BLK, DIM), imap_x),
        scratch_shapes=[pltpu.VMEM((8, SD), jnp.float32)],
    )
    return pl.pallas_call(
        _moe_body,
        grid_spec=grid_spec,
        out_shape=jax.ShapeDtypeStruct((NBLK, BLK, DIM), jnp.float32),
        compiler_params=pltpu.CompilerParams(
            dimension_semantics=("arbitrary",)),
    )(be_flat, used_flat, vcnt_flat, fstart_flat, x_sorted, W_c,
      W_sel_out, W_out, dp3)


def _sc_mesh():
    return plsc.VectorSubcoreMesh(core_axis_name="c", subcore_axis_name="s")


def _sc_scatter(x_flat, idx3):
    @functools.partial(
        pl.kernel,
        mesh=_sc_mesh(),
        out_type=jax.ShapeDtypeStruct((PADS, DIM), jnp.float32),
        scratch_types=[
            pltpu.VMEM((_NCH, _CH), jnp.int32),
            pltpu.VMEM((_CH, DIM), jnp.float32),
            pltpu.VMEM((_CH, DIM), jnp.float32),
            pltpu.SemaphoreType.DMA,
            pltpu.SemaphoreType.DMA,
            pltpu.SemaphoreType.DMA,
            pltpu.SemaphoreType.DMA,
        ],
    )
    def k(x_hbm, idx_hbm, out_hbm, idxv, rv0, rv1, sr0, sr1, sw0, sw1):
        wid = jax.lax.axis_index("s") * _SC_NC + jax.lax.axis_index("c")
        pltpu.sync_copy(idx_hbm.at[wid], idxv)
        bufs = (rv0, rv1)
        srs = (sr0, sr1)
        sws = (sw0, sw1)
        base = wid * _RPW
        rd0 = pltpu.async_copy(x_hbm.at[pl.ds(base, _CH)], rv0, sr0)
        reads = [rd0, None]
        writes = [None, None]
        for j in range(_NCH):
            p = j % 2
            q = (j + 1) % 2
            reads[p].wait()
            if j + 1 < _NCH:
                if writes[q] is not None:
                    writes[q].wait()
                reads[q] = pltpu.async_copy(
                    x_hbm.at[pl.ds(base + (j + 1) * _CH, _CH)], bufs[q],
                    srs[q])
            writes[p] = pltpu.async_copy(
                bufs[p], out_hbm.at[idxv.at[jnp.int32(j)]], sws[p])
        writes[0].wait()
        writes[1].wait()

    return k(x_flat, idx3)


def _sc_gather(src_flat, idx3):
    @functools.partial(
        pl.kernel,
        mesh=_sc_mesh(),
        out_type=jax.ShapeDtypeStruct((B * S, DIM), jnp.float32),
        scratch_types=[
            pltpu.VMEM((_NCH, _CH), jnp.int32),
            pltpu.VMEM((_CH, DIM), jnp.float32),
            pltpu.VMEM((_CH, DIM), jnp.float32),
            pltpu.SemaphoreType.DMA,
            pltpu.SemaphoreType.DMA,
            pltpu.SemaphoreType.DMA,
            pltpu.SemaphoreType.DMA,
        ],
    )
    def k(src_hbm, idx_hbm, out_hbm, idxv, rv0, rv1, sr0, sr1, sw0, sw1):
        wid = jax.lax.axis_index("s") * _SC_NC + jax.lax.axis_index("c")
        pltpu.sync_copy(idx_hbm.at[wid], idxv)
        bufs = (rv0, rv1)
        srs = (sr0, sr1)
        sws = (sw0, sw1)
        base = wid * _RPW
        rd0 = pltpu.async_copy(src_hbm.at[idxv.at[jnp.int32(0)]], rv0, sr0)
        reads = [rd0, None]
        writes = [None, None]
        for j in range(_NCH):
            p = j % 2
            q = (j + 1) % 2
            reads[p].wait()
            if j + 1 < _NCH:
                if writes[q] is not None:
                    writes[q].wait()
                reads[q] = pltpu.async_copy(
                    src_hbm.at[idxv.at[jnp.int32(j + 1)]], bufs[q], srs[q])
            writes[p] = pltpu.async_copy(
                bufs[p], out_hbm.at[pl.ds(base + j * _CH, _CH)], sws[p])
        writes[0].wait()
        writes[1].wait()

    return k(src_flat, idx3)


def kernel(x, token_ids, W_in, W_sel_in, W_sel_out, W_out, d_param):
    tok_t = token_ids.astype(jnp.int32).T
    gpos_t, be, used, vcnt, fstart = _plan(tok_t)

    idx3 = gpos_t.T.reshape(_NW, _NCH, _CH)
    x_sorted = _sc_scatter(x.reshape(B * S, DIM), idx3)

    out_sorted = _moe(
        be.reshape(NBLK),
        used.reshape(NBLK),
        vcnt.reshape(NBLK),
        fstart.reshape(NBLK),
        x_sorted.reshape(NBLK, BLK, DIM),
        jnp.concatenate([W_in, W_sel_in], axis=2).astype(jnp.bfloat16),
        W_sel_out.astype(jnp.bfloat16),
        W_out.astype(jnp.bfloat16),
        d_param.reshape(E, 1, SD),
    )

    out = _sc_gather(out_sorted.reshape(PADS, DIM), idx3)
    return out.reshape(B, S, DIM)
